# baseline (speedup 1.0000x reference)
import functools

import jax
import jax.numpy as jnp
from jax import lax
from jax.experimental import pallas as pl
from jax.experimental.pallas import tpu as pltpu
from jax.experimental.pallas import tpu_sc as plsc

B, N, C = 16, 100000, 3
NPOINT = 4096

NUM_CORES = 2
NUM_SUBCORES = 16
NW = NUM_CORES * NUM_SUBCORES
ROWS_PER_W = (B * NPOINT) // NW
CHUNK = 128
NCHUNK = ROWS_PER_W // CHUNK
LANES = 16

_mesh = plsc.VectorSubcoreMesh(core_axis_name="c", subcore_axis_name="s")


@functools.partial(
    pl.kernel,
    mesh=_mesh,
    out_type=jax.ShapeDtypeStruct((B * NPOINT, C), jnp.float32),
    scratch_types=[
        pltpu.VMEM((NCHUNK, CHUNK), jnp.int32),
        pltpu.VMEM((ROWS_PER_W, C), jnp.float32),
        pltpu.SemaphoreType.DMA,
    ],
    compiler_params=pltpu.CompilerParams(use_tc_tiling_on_sc=False),
)
def _gather_sc(table_hbm, idx_hbm, out_hbm, idx_v, rows_v, sem):
    wid = lax.axis_index("s") * NUM_CORES + lax.axis_index("c")
    batch = wid // (NPOINT // ROWS_PER_W)
    base = batch * N

    pltpu.sync_copy(idx_hbm.at[pl.ds(wid * NCHUNK, NCHUNK)], idx_v)

    for i in range(NCHUNK):
        for k in range(CHUNK // LANES):
            sl = (i, pl.ds(k * LANES, LANES))
            idx_v[sl] = idx_v[sl] + base

    copies = []
    for j in range(NCHUNK):
        copies.append(
            pltpu.async_copy(
                table_hbm.at[idx_v.at[j]],
                rows_v.at[pl.ds(j * CHUNK, CHUNK)],
                sem,
            )
        )
    for cp in copies:
        cp.wait()

    pltpu.sync_copy(rows_v, out_hbm.at[pl.ds(wid * ROWS_PER_W, ROWS_PER_W)])


def kernel(xyz, point_indices):
    table = xyz.reshape(B * N, C)
    idx = point_indices.astype(jnp.int32).reshape((B * NPOINT) // CHUNK, CHUNK)
    out = _gather_sc(table, idx)
    return out.reshape(B, NPOINT, C)


# speedup vs baseline: 1.0813x; 1.0813x over previous
import functools

import jax
import jax.numpy as jnp
from jax import lax
from jax.experimental import pallas as pl
from jax.experimental.pallas import tpu as pltpu
from jax.experimental.pallas import tpu_sc as plsc

B, N, C = 16, 100000, 3
NPOINT = 4096

NUM_CORES = 2
NUM_SUBCORES = 16
NW = NUM_CORES * NUM_SUBCORES
PTS_PER_W = (B * NPOINT) // NW
CHUNK = 128
NCHUNK = PTS_PER_W // CHUNK
LANES = 16
D = 8
NROWS = (B * N * C) // D
OUT_WORDS_W = PTS_PER_W * C

_mesh = plsc.VectorSubcoreMesh(core_axis_name="c", subcore_axis_name="s")


@functools.partial(
    pl.kernel,
    mesh=_mesh,
    out_type=jax.ShapeDtypeStruct((B * NPOINT * C,), jnp.float32),
    scratch_types=[
        pltpu.VMEM((NCHUNK, CHUNK), jnp.int32),
        pltpu.VMEM((NCHUNK, CHUNK), jnp.int32),
        pltpu.VMEM((PTS_PER_W,), jnp.int32),
        pltpu.VMEM((2 * PTS_PER_W, D), jnp.float32),
        pltpu.VMEM((OUT_WORDS_W,), jnp.float32),
        pltpu.SemaphoreType.DMA,
    ],
    compiler_params=pltpu.CompilerParams(
        use_tc_tiling_on_sc=False, needs_layout_passes=False
    ),
)
def _gather_sc(table_hbm, idx_hbm, out_hbm, ra_v, rb_v, off_v, win_v, rows_v, sem):
    wid = lax.axis_index("s") * NUM_CORES + lax.axis_index("c")
    batch = wid // (NPOINT // PTS_PER_W)
    base = batch * N

    pltpu.sync_copy(idx_hbm.at[pl.ds(wid * NCHUNK, NCHUNK)], ra_v)

    for i in range(NCHUNK):
        for k in range(CHUNK // LANES):
            sl = (i, pl.ds(k * LANES, LANES))
            p = (ra_v[sl] + base) * 3
            r = lax.shift_right_logical(p, 3)
            ra_v[sl] = r
            rb_v[sl] = jnp.minimum(r + 1, NROWS - 1)
            off_v[pl.ds((i * (CHUNK // LANES) + k) * LANES, LANES)] = (
                lax.bitwise_and(p, 7)
            )

    copies = []
    for j in range(NCHUNK):
        copies.append(
            pltpu.async_copy(
                table_hbm.at[ra_v.at[j]],
                win_v.at[pl.ds(j * CHUNK, CHUNK)],
                sem,
            )
        )
        copies.append(
            pltpu.async_copy(
                table_hbm.at[rb_v.at[j]],
                win_v.at[pl.ds(PTS_PER_W + j * CHUNK, CHUNK)],
                sem,
            )
        )
    for cp in copies:
        cp.wait()

    lane = lax.iota(jnp.int32, LANES)
    phase_t = []
    phase_c = []
    for q in range(3):
        x = lane + q * LANES
        t = lax.shift_right_logical(x * 21846, 16)
        phase_t.append(t)
        phase_c.append(x - 3 * t)

    def body(blk, carry):
        tbase = blk * LANES
        obase = blk * (3 * LANES)
        for q in range(3):
            tt = tbase + phase_t[q]
            offs = plsc.load_gather(off_v, [tt])
            w = offs + phase_c[q]
            in_b = w >= D
            row = tt + jnp.where(in_b, PTS_PER_W, 0)
            col = lax.bitwise_and(w, D - 1)
            val = plsc.load_gather(win_v, [row, col])
            rows_v[pl.ds(obase + q * LANES, LANES)] = val
        return carry

    lax.fori_loop(0, PTS_PER_W // LANES, body, 0)

    pltpu.sync_copy(rows_v, out_hbm.at[pl.ds(wid * OUT_WORDS_W, OUT_WORDS_W)])


def kernel(xyz, point_indices):
    table = xyz.reshape(NROWS, D)
    idx = point_indices.astype(jnp.int32).reshape((B * NPOINT) // CHUNK, CHUNK)
    out = _gather_sc(table, idx)
    return out.reshape(B, NPOINT, C)


# speedup vs baseline: 159.3969x; 147.4138x over previous
import functools

import jax
import jax.numpy as jnp
from jax import lax
from jax.experimental import pallas as pl
from jax.experimental.pallas import tpu as pltpu
from jax.experimental.pallas import tpu_sc as plsc

B, N, C = 16, 100000, 3
NPOINT = 4096

NUM_CORES = 2
NUM_SUBCORES = 16
NW = NUM_CORES * NUM_SUBCORES
NTASK = C * B
IDX_ROWS = NPOINT // 128
LANES = 16

_mesh = plsc.VectorSubcoreMesh(core_axis_name="c", subcore_axis_name="s")


@functools.partial(
    pl.kernel,
    mesh=_mesh,
    out_type=jax.ShapeDtypeStruct((C * B * IDX_ROWS, 128), jnp.float32),
    scratch_types=[
        pltpu.VMEM((N,), jnp.float32),
        pltpu.VMEM((IDX_ROWS, 128), jnp.int32),
        pltpu.VMEM((IDX_ROWS, 128), jnp.float32),
    ],
    compiler_params=pltpu.CompilerParams(
        use_tc_tiling_on_sc=True, needs_layout_passes=False
    ),
)
def _gather_sc(xyz_t, idx_hbm, out_hbm, plane_v, idx_v, outp_v):
    wid = lax.axis_index("s") * NUM_CORES + lax.axis_index("c")

    def do_task(t):
        c = t // B
        b = t % B
        pltpu.sync_copy(xyz_t.at[c, b], plane_v)
        pltpu.sync_copy(idx_hbm.at[pl.ds(b * IDX_ROWS, IDX_ROWS)], idx_v)

        def body(j, carry):
            row = j // (128 // LANES)
            col = (j % (128 // LANES)) * LANES
            iv = idx_v[row, pl.ds(col, LANES)]
            outp_v[row, pl.ds(col, LANES)] = plsc.load_gather(plane_v, [iv])
            return carry

        lax.fori_loop(0, NPOINT // LANES, body, 0)
        pltpu.sync_copy(outp_v, out_hbm.at[pl.ds(t * IDX_ROWS, IDX_ROWS)])

    do_task(wid)

    @pl.when(wid < NTASK - NW)
    def _():
        do_task(wid + NW)


def kernel(xyz, point_indices):
    xyz_t = jnp.transpose(xyz, (2, 0, 1))
    idx = point_indices.astype(jnp.int32).reshape(B * NPOINT // 128, 128)
    out = _gather_sc(xyz_t, idx)
    return jnp.transpose(out.reshape(C, B, NPOINT), (1, 2, 0))


# speedup vs baseline: 170.8293x; 1.0717x over previous
import functools

import jax
import jax.numpy as jnp
from jax import lax
from jax.experimental import pallas as pl
from jax.experimental.pallas import tpu as pltpu
from jax.experimental.pallas import tpu_sc as plsc

B, N, C = 16, 100000, 3
NPOINT = 4096

NUM_CORES = 2
NUM_SUBCORES = 16
NW = NUM_CORES * NUM_SUBCORES
NTASK = C * B
IDX_ROWS = NPOINT // 128
LANES = 16

_mesh = plsc.VectorSubcoreMesh(core_axis_name="c", subcore_axis_name="s")


@functools.partial(
    pl.kernel,
    mesh=_mesh,
    out_type=jax.ShapeDtypeStruct((C, B, NPOINT), jnp.float32),
    scratch_types=[
        pltpu.VMEM((N,), jnp.float32),
        pltpu.VMEM((NPOINT,), jnp.int32),
        pltpu.VMEM((NPOINT,), jnp.float32),
    ],
    compiler_params=pltpu.CompilerParams(
        use_tc_tiling_on_sc=True, needs_layout_passes=False
    ),
)
def _gather_sc(xyz_t, idx_hbm, out_hbm, plane_v, idx_v, outp_v):
    wid = lax.axis_index("s") * NUM_CORES + lax.axis_index("c")

    def do_task(t, carry):
        c = t // B
        b = t - c * B
        pltpu.sync_copy(xyz_t.at[c, b], plane_v)
        pltpu.sync_copy(idx_hbm.at[b], idx_v)

        def body(j, carry2):
            col = j * LANES
            iv = idx_v[pl.ds(col, LANES)]
            outp_v[pl.ds(col, LANES)] = plsc.load_gather(plane_v, [iv])
            return carry2

        lax.fori_loop(0, NPOINT // LANES, body, 0)
        pltpu.sync_copy(outp_v, out_hbm.at[c, b])
        return carry

    ntasks = jnp.where(wid < NTASK - NW, 2, 1)
    lax.fori_loop(0, ntasks, lambda i, cr: do_task(wid + i * NW, cr), 0)


def kernel(xyz, point_indices):
    xyz_t = jnp.transpose(xyz, (2, 0, 1))
    idx = point_indices.astype(jnp.int32)
    out = _gather_sc(xyz_t, idx)
    return jnp.transpose(out, (1, 2, 0))


# speedup vs baseline: 180.7350x; 1.0580x over previous
import functools

import jax
import jax.numpy as jnp
from jax import lax
from jax.experimental import pallas as pl
from jax.experimental.pallas import tpu as pltpu
from jax.experimental.pallas import tpu_sc as plsc

B, N, C = 16, 100000, 3
NPOINT = 4096

NUM_CORES = 2
NUM_SUBCORES = 16
NW = NUM_CORES * NUM_SUBCORES
NTASK = C * B
IDX_ROWS = NPOINT // 128
LANES = 16

_mesh = plsc.VectorSubcoreMesh(core_axis_name="c", subcore_axis_name="s")


@functools.partial(
    pl.kernel,
    mesh=_mesh,
    out_type=jax.ShapeDtypeStruct((C, B, NPOINT), jnp.float32),
    scratch_types=[
        pltpu.VMEM((N,), jnp.float32),
        pltpu.VMEM((NPOINT,), jnp.int32),
        pltpu.VMEM((NPOINT,), jnp.float32),
        pltpu.SemaphoreType.DMA,
    ],
    compiler_params=pltpu.CompilerParams(
        use_tc_tiling_on_sc=True, needs_layout_passes=False
    ),
)
def _gather_sc(xyz_t, idx_hbm, out_hbm, plane_v, idx_v, outp_v, sem0):
    wid = lax.axis_index("s") * NUM_CORES + lax.axis_index("c")

    def do_task(t, carry):
        c = t // B
        b = t - c * B
        cp = pltpu.async_copy(xyz_t.at[c, b], plane_v, sem0)
        pltpu.sync_copy(idx_hbm.at[b], idx_v)
        cp.wait()

        def body(j, carry2):
            for u in range(4):
                col = j * (4 * LANES) + u * LANES
                iv = idx_v[pl.ds(col, LANES)]
                outp_v[pl.ds(col, LANES)] = plsc.load_gather(plane_v, [iv])
            return carry2

        lax.fori_loop(0, NPOINT // (4 * LANES), body, 0)
        pltpu.sync_copy(outp_v, out_hbm.at[c, b])
        return carry

    ntasks = jnp.where(wid < NTASK - NW, 2, 1)
    lax.fori_loop(0, ntasks, lambda i, cr: do_task(wid + i * NW, cr), 0)


def kernel(xyz, point_indices):
    xyz_t = jnp.transpose(xyz, (2, 0, 1))
    idx = point_indices.astype(jnp.int32)
    out = _gather_sc(xyz_t, idx)
    return jnp.transpose(out, (1, 2, 0))


# speedup vs baseline: 181.3547x; 1.0034x over previous
import functools

import jax
import jax.numpy as jnp
from jax import lax
from jax.experimental import pallas as pl
from jax.experimental.pallas import tpu as pltpu
from jax.experimental.pallas import tpu_sc as plsc

B, N, C = 16, 100000, 3
NPOINT = 4096

NUM_CORES = 2
NUM_SUBCORES = 16
NW = NUM_CORES * NUM_SUBCORES
NTASK = C * B
IDX_ROWS = NPOINT // 128
LANES = 16

_mesh = plsc.VectorSubcoreMesh(core_axis_name="c", subcore_axis_name="s")


@functools.partial(
    pl.kernel,
    mesh=_mesh,
    out_type=jax.ShapeDtypeStruct((C, B, NPOINT), jnp.float32),
    scratch_types=[
        pltpu.VMEM((N,), jnp.float32),
        pltpu.VMEM((NPOINT,), jnp.int32),
        pltpu.VMEM((NPOINT,), jnp.float32),
        pltpu.SemaphoreType.DMA,
    ],
    compiler_params=pltpu.CompilerParams(
        use_tc_tiling_on_sc=True,
        needs_layout_passes=False,
        skip_device_barrier=True,
        disable_bounds_checks=True,
    ),
)
def _gather_sc(xyz_t, idx_hbm, out_hbm, plane_v, idx_v, outp_v, sem0):
    wid = lax.axis_index("s") * NUM_CORES + lax.axis_index("c")

    def do_task(t, carry):
        c = t // B
        b = t - c * B
        cp = pltpu.async_copy(xyz_t.at[c, b], plane_v, sem0)
        pltpu.sync_copy(idx_hbm.at[b], idx_v)
        cp.wait()

        def body(j, carry2):
            for u in range(4):
                col = j * (4 * LANES) + u * LANES
                iv = idx_v[pl.ds(col, LANES)]
                outp_v[pl.ds(col, LANES)] = plsc.load_gather(plane_v, [iv])
            return carry2

        lax.fori_loop(0, NPOINT // (4 * LANES), body, 0)
        pltpu.sync_copy(outp_v, out_hbm.at[c, b])
        return carry

    ntasks = jnp.where(wid < NTASK - NW, 2, 1)
    lax.fori_loop(0, ntasks, lambda i, cr: do_task(wid + i * NW, cr), 0)


def kernel(xyz, point_indices):
    xyz_t = jnp.transpose(xyz, (2, 0, 1))
    idx = point_indices.astype(jnp.int32)
    out = _gather_sc(xyz_t, idx)
    return jnp.transpose(out, (1, 2, 0))


# speedup vs baseline: 181.4247x; 1.0004x over previous
import functools

import jax
import jax.numpy as jnp
from jax import lax
from jax.experimental import pallas as pl
from jax.experimental.pallas import tpu as pltpu
from jax.experimental.pallas import tpu_sc as plsc

B, N, C = 16, 100000, 3
NPOINT = 4096

NUM_CORES = 2
NUM_SUBCORES = 16
NW = NUM_CORES * NUM_SUBCORES
NTASK = C * B
IDX_ROWS = NPOINT // 128
LANES = 16

_mesh = plsc.VectorSubcoreMesh(core_axis_name="c", subcore_axis_name="s")


@functools.partial(
    pl.kernel,
    mesh=_mesh,
    out_type=jax.ShapeDtypeStruct((C, B, NPOINT), jnp.float32),
    scratch_types=[
        pltpu.VMEM((N,), jnp.float32),
        pltpu.VMEM((NPOINT,), jnp.int32),
        pltpu.VMEM((NPOINT,), jnp.float32),
        pltpu.SemaphoreType.DMA,
    ],
    compiler_params=pltpu.CompilerParams(
        use_tc_tiling_on_sc=True, needs_layout_passes=False
    ),
)
def _gather_sc(xyz_t, idx_hbm, out_hbm, plane_v, idx_v, outp_v, sem0):
    wid = lax.axis_index("s") * NUM_CORES + lax.axis_index("c")

    def do_task(t, carry):
        c = t // B
        b = t - c * B
        cp = pltpu.async_copy(xyz_t.at[c, b], plane_v, sem0)
        pltpu.sync_copy(idx_hbm.at[b], idx_v)
        cp.wait()

        def body(j, carry2):
            for u in range(4):
                col = j * (4 * LANES) + u * LANES
                iv = idx_v[pl.ds(col, LANES)]
                outp_v[pl.ds(col, LANES)] = plsc.load_gather(plane_v, [iv])
            return carry2

        lax.fori_loop(0, NPOINT // (4 * LANES), body, 0)
        pltpu.sync_copy(outp_v, out_hbm.at[c, b])
        return carry

    ntasks = jnp.where(wid < NTASK - NW, 2, 1)
    lax.fori_loop(0, ntasks, lambda i, cr: do_task(wid + i * NW, cr), 0)


def kernel(xyz, point_indices):
    xyz_t = jnp.transpose(xyz, (2, 0, 1))
    idx = point_indices.astype(jnp.int32)
    out = _gather_sc(xyz_t, idx)
    return jnp.transpose(out, (1, 2, 0))


# speedup vs baseline: 181.8689x; 1.0024x over previous
import functools

import jax
import jax.numpy as jnp
from jax import lax
from jax.experimental import pallas as pl
from jax.experimental.pallas import tpu as pltpu
from jax.experimental.pallas import tpu_sc as plsc

B, N, C = 16, 100000, 3
NPOINT = 4096

NUM_CORES = 2
NUM_SUBCORES = 16
NW = NUM_CORES * NUM_SUBCORES
NTASK = C * B
LANES = 16

@functools.cache
def _build_gather_sc():
    mesh = plsc.VectorSubcoreMesh(core_axis_name="c", subcore_axis_name="s")

    @functools.partial(
        pl.kernel,
        mesh=mesh,
        out_type=jax.ShapeDtypeStruct((C, B, NPOINT), jnp.float32),
        scratch_types=[
            pltpu.VMEM((N,), jnp.float32),
            pltpu.VMEM((NPOINT,), jnp.int32),
            pltpu.VMEM((NPOINT,), jnp.float32),
            pltpu.SemaphoreType.DMA,
        ],
        compiler_params=pltpu.CompilerParams(
            use_tc_tiling_on_sc=True, needs_layout_passes=False
        ),
    )
    def _gather_sc(xyz_t, idx_hbm, out_hbm, plane_v, idx_v, outp_v, sem0):
        wid = lax.axis_index("s") * NUM_CORES + lax.axis_index("c")

        def do_task(t, carry):
            c = t // B
            b = t - c * B
            cp = pltpu.async_copy(xyz_t.at[c, b], plane_v, sem0)
            pltpu.sync_copy(idx_hbm.at[b], idx_v)
            cp.wait()

            def body(j, carry2):
                for u in range(4):
                    col = j * (4 * LANES) + u * LANES
                    iv = idx_v[pl.ds(col, LANES)]
                    outp_v[pl.ds(col, LANES)] = plsc.load_gather(plane_v, [iv])
                return carry2

            lax.fori_loop(0, NPOINT // (4 * LANES), body, 0)
            pltpu.sync_copy(outp_v, out_hbm.at[c, b])
            return carry

        ntasks = jnp.where(wid < NTASK - NW, 2, 1)
        lax.fori_loop(0, ntasks, lambda i, cr: do_task(wid + i * NW, cr), 0)

    return _gather_sc


def kernel(xyz, point_indices):
    xyz_t = jnp.transpose(xyz, (2, 0, 1))
    idx = point_indices.astype(jnp.int32)
    out = _build_gather_sc()(xyz_t, idx)
    return jnp.transpose(out, (1, 2, 0))
